# baseline (speedup 1.0000x reference)
import functools

import jax
import jax.numpy as jnp
from jax import lax
from jax.experimental import pallas as pl
from jax.experimental.pallas import tpu as pltpu
from jax.experimental.pallas import tpu_sc as plsc

PADDING_IDX = 2
B, T = 4, 4096
N = B * T
D = 768
L = 16
NC, NS = 2, 16
NW = NC * NS
PER_W = N // NW
TCH = 32
NCHUNK = PER_W // TCH
NBUF = 4
NJG = 3
JGC = 16
JGW = JGC * L

_mesh = plsc.VectorSubcoreMesh(core_axis_name="c", subcore_axis_name="s")

_DNUMS = lax.GatherDimensionNumbers(
    offset_dims=(), collapsed_slice_dims=(0,), start_index_map=(0,)
)


def _bcast(v, p):
    idx = jnp.full((L, 1), p, jnp.int32)
    return lax.gather(
        v, idx, _DNUMS, (1,), mode=lax.GatherScatterMode.PROMISE_IN_BOUNDS
    )


@functools.partial(
    pl.kernel,
    mesh=_mesh,
    out_type=jax.ShapeDtypeStruct((N, D), jnp.float32),
    scratch_types=[
        pltpu.VMEM((PER_W,), jnp.int32),
        pltpu.VMEM((PER_W,), jnp.int32),
        pltpu.VMEM((3, D), jnp.float32),
        pltpu.VMEM((NBUF, TCH, D), jnp.float32),
        pltpu.SemaphoreType.DMA,
        pltpu.SemaphoreType.DMA,
        pltpu.SemaphoreType.DMA,
        pltpu.SemaphoreType.DMA,
        pltpu.SemaphoreType.DMA,
    ],
)
def _sc_embed(ids_hbm, mask_hbm, table_hbm, out_hbm,
              ids_v, msk_v, table_v, rows_v, sem_s, sem_o0, sem_o1, sem_o2, sem_o3):
    wid = lax.axis_index("s") * NC + lax.axis_index("c")
    base = wid * PER_W

    c_tab = pltpu.async_copy(table_hbm, table_v, sem_s)
    c_ids = pltpu.async_copy(ids_hbm.at[pl.ds(base, PER_W)], ids_v, sem_s)
    c_msk = pltpu.async_copy(mask_hbm.at[pl.ds(base, PER_W)], msk_v, sem_s)
    c_tab.wait()
    c_ids.wait()
    c_msk.wait()

    one = jnp.full((L,), 1.0, jnp.float32)
    zero = jnp.full((L,), 0.0, jnp.float32)
    sem_o = (sem_o0, sem_o1, sem_o2, sem_o3)

    def build(ci, buf):
        for jg in range(NJG):
            w0s = [table_v[0, pl.ds(jg * JGW + k * L, L)] for k in range(JGC)]
            w1s = [table_v[1, pl.ds(jg * JGW + k * L, L)] for k in range(JGC)]

            def tg_body(tg, carry):
                t0 = ci * TCH + tg * L
                idv = ids_v[pl.ds(t0, L)]
                valid = msk_v[pl.ds(t0, L)] != 0
                av = jnp.where(valid & (idv == 0), one, zero)
                bv = jnp.where(valid & (idv == 1), one, zero)
                for p in range(L):
                    abc = _bcast(av, p)
                    bbc = _bcast(bv, p)
                    tloc = tg * L + p
                    for k in range(JGC):
                        rows_v[buf, tloc, pl.ds(jg * JGW + k * L, L)] = (
                            w0s[k] * abc + w1s[k] * bbc
                        )
                return carry

            lax.fori_loop(0, TCH // L, tg_body, 0)

    def pair_body(cp, carry):
        for buf in range(NBUF):
            ci = cp * NBUF + buf

            @pl.when(cp > 0)
            def _wait():
                pltpu.make_async_copy(
                    rows_v.at[buf], out_hbm.at[pl.ds(base, TCH)], sem_o[buf]
                ).wait()

            build(ci, buf)
            pltpu.async_copy(
                rows_v.at[buf],
                out_hbm.at[pl.ds(base + ci * TCH, TCH)],
                sem_o[buf],
            )
        return carry

    lax.fori_loop(0, NCHUNK // NBUF, pair_body, 0)

    for buf in range(NBUF):
        pltpu.make_async_copy(
            rows_v.at[buf], out_hbm.at[pl.ds(base, TCH)], sem_o[buf]
        ).wait()


def kernel(token_types_id, attn_mask, W):
    ids = token_types_id.reshape(N).astype(jnp.int32)
    msk = attn_mask.reshape(N).astype(jnp.int32)
    out = _sc_embed(ids, msk, W)
    return out.reshape(B, T, D)


# speedup vs baseline: 1.2561x; 1.2561x over previous
import functools

import jax
import jax.numpy as jnp
from jax import lax
from jax.experimental import pallas as pl
from jax.experimental.pallas import tpu as pltpu
from jax.experimental.pallas import tpu_sc as plsc

PADDING_IDX = 2
B, T = 4, 4096
N = B * T
D = 768
L = 16
NC, NS = 2, 16
NW = NC * NS
PER_W = N // NW
TCH = 64
NCHUNK = PER_W // TCH
NBUF = 2
NJG = 6
JGC = 8
JGW = JGC * L

_mesh = plsc.VectorSubcoreMesh(core_axis_name="c", subcore_axis_name="s")

_DNUMS = lax.GatherDimensionNumbers(
    offset_dims=(), collapsed_slice_dims=(0,), start_index_map=(0,)
)


def _bcast(v, p):
    idx = jnp.full((L, 1), p, jnp.int32)
    return lax.gather(
        v, idx, _DNUMS, (1,), mode=lax.GatherScatterMode.PROMISE_IN_BOUNDS
    )


@functools.partial(
    pl.kernel,
    mesh=_mesh,
    out_type=jax.ShapeDtypeStruct((N, D), jnp.float32),
    scratch_types=[
        pltpu.VMEM((PER_W,), jnp.int32),
        pltpu.VMEM((PER_W,), jnp.int32),
        pltpu.VMEM((3, D), jnp.float32),
        pltpu.VMEM((NBUF, TCH, D), jnp.float32),
        pltpu.SemaphoreType.DMA,
        pltpu.SemaphoreType.DMA,
        pltpu.SemaphoreType.DMA,
    ],
)
def _sc_embed(ids_hbm, mask_hbm, table_hbm, out_hbm,
              ids_v, msk_v, table_v, rows_v, sem_s, sem_o0, sem_o1):
    wid = lax.axis_index("s") * NC + lax.axis_index("c")
    base = wid * PER_W

    c_tab = pltpu.async_copy(table_hbm, table_v, sem_s)
    c_ids = pltpu.async_copy(ids_hbm.at[pl.ds(base, PER_W)], ids_v, sem_s)
    c_msk = pltpu.async_copy(mask_hbm.at[pl.ds(base, PER_W)], msk_v, sem_s)
    c_tab.wait()
    c_ids.wait()
    c_msk.wait()

    one = jnp.full((L,), 1.0, jnp.float32)
    zero = jnp.full((L,), 0.0, jnp.float32)
    sem_o = (sem_o0, sem_o1)

    def build(ci, buf):
        for jg in range(NJG):
            w1s = [table_v[1, pl.ds(jg * JGW + k * L, L)] for k in range(JGC)]
            dls = [table_v[0, pl.ds(jg * JGW + k * L, L)] - w1s[k]
                   for k in range(JGC)]

            def tg_body(tg, carry):
                t0 = ci * TCH + tg * L
                idv = ids_v[pl.ds(t0, L)]
                a0v = jnp.where(idv == 0, one, zero)
                validf = jnp.where(msk_v[pl.ds(t0, L)] != 0, one, zero)
                for p in range(L):
                    a0bc = _bcast(a0v, p)
                    vbc = _bcast(validf, p)
                    tloc = tg * L + p
                    for k in range(JGC):
                        rows_v[buf, tloc, pl.ds(jg * JGW + k * L, L)] = (
                            (w1s[k] + dls[k] * a0bc) * vbc
                        )
                return carry

            lax.fori_loop(0, TCH // L, tg_body, 0)

    def pair_body(cp, carry):
        for buf in range(NBUF):
            ci = cp * NBUF + buf

            @pl.when(cp > 0)
            def _wait():
                pltpu.make_async_copy(
                    rows_v.at[buf], out_hbm.at[pl.ds(base, TCH)], sem_o[buf]
                ).wait()

            build(ci, buf)
            pltpu.async_copy(
                rows_v.at[buf],
                out_hbm.at[pl.ds(base + ci * TCH, TCH)],
                sem_o[buf],
            )
        return carry

    lax.fori_loop(0, NCHUNK // NBUF, pair_body, 0)

    for buf in range(NBUF):
        pltpu.make_async_copy(
            rows_v.at[buf], out_hbm.at[pl.ds(base, TCH)], sem_o[buf]
        ).wait()


def kernel(token_types_id, attn_mask, W):
    ids = token_types_id.reshape(N).astype(jnp.int32)
    msk = attn_mask.reshape(N).astype(jnp.int32)
    out = _sc_embed(ids, msk, W)
    return out.reshape(B, T, D)
